# baseline (speedup 1.0000x reference)
import jax
import jax.numpy as jnp
from jax import lax
from jax.experimental import pallas as pl
from jax.experimental.pallas import tpu as pltpu
from jax.experimental.pallas import tpu_sc as plsc

_B, _S, _D = 4, 8192, 1024
_NC, _NS = 2, 16
_NW = _NC * _NS
_RPW = _B * _S // _NW
_WPB = _S // _RPW
_C = 32
_NCHUNK = _RPW // _C
_LANES = 16


def _body(x_hbm, idx_hbm, out_hbm, idx_v, buf0, buf1, buf2,
          gsem0, gsem1, gsem2, ssem0, ssem1, ssem2):
    wid = lax.axis_index("s") * _NC + lax.axis_index("c")
    b = wid // _WPB
    seq_lo = (wid % _WPB) * _RPW
    out_lo = wid * _RPW

    pltpu.sync_copy(idx_hbm.at[pl.ds(seq_lo, _RPW)], idx_v)
    row_off = b * _S
    for j in range(_RPW // _LANES):
        sl = pl.ds(j * _LANES, _LANES)
        idx_v[sl] = idx_v[sl] + row_off

    bufs = (buf0, buf1, buf2)
    gsems = (gsem0, gsem1, gsem2)
    ssems = (ssem0, ssem1, ssem2)

    def _gather_desc(g, par):
        off = pl.multiple_of(g * _C, _C)
        src = x_hbm.at[idx_v.at[pl.ds(off, _C)]]
        return pltpu.make_async_copy(src, bufs[par], gsems[par])

    def _store_desc(g, par):
        dst = out_hbm.at[pl.ds(out_lo + g * _C, _C)]
        return pltpu.make_async_copy(bufs[par], dst, ssems[par])

    _gather_desc(0, 0).start()
    _gather_desc(1, 1).start()
    _gather_desc(2, 2).start()

    for h in range(2):
        _gather_desc(h, h).wait()
        _store_desc(h, h).start()

    @pl.loop(2, _NCHUNK - 3, step=3)
    def _ring(h0):
        for j in range(3):
            h = h0 + j
            par = (2 + j) % 3
            _gather_desc(h, par).wait()
            _store_desc(h, par).start()
            q = (par + 1) % 3
            _store_desc(h - 2, q).wait()
            _gather_desc(h + 1, q).start()

    for h in range(_NCHUNK - 3, _NCHUNK - 1):
        par = h % 3
        q = (par + 1) % 3
        _gather_desc(h, par).wait()
        _store_desc(h, par).start()
        _store_desc(h - 2, q).wait()
        _gather_desc(h + 1, q).start()
    h = _NCHUNK - 1
    _gather_desc(h, h % 3).wait()
    _store_desc(h, h % 3).start()
    for g in range(_NCHUNK - 3, _NCHUNK):
        _store_desc(g, g % 3).wait()


def _build():
    mesh = plsc.VectorSubcoreMesh(
        core_axis_name="c", subcore_axis_name="s",
        num_cores=_NC, num_subcores=_NS)
    return pl.kernel(
        _body,
        out_type=jax.ShapeDtypeStruct((_B * _S, _D), jnp.float32),
        mesh=mesh,
        scratch_types=[
            pltpu.VMEM((_RPW,), jnp.int32),
            pltpu.VMEM((_C, _D), jnp.float32),
            pltpu.VMEM((_C, _D), jnp.float32),
            pltpu.VMEM((_C, _D), jnp.float32),
            pltpu.SemaphoreType.DMA,
            pltpu.SemaphoreType.DMA,
            pltpu.SemaphoreType.DMA,
            pltpu.SemaphoreType.DMA,
            pltpu.SemaphoreType.DMA,
            pltpu.SemaphoreType.DMA,
        ],
    )


def kernel(x, index):
    B, S, D = x.shape
    assert (B, S, D) == (_B, _S, _D)
    out = _build()(x.reshape(B * S, D), index.astype(jnp.int32))
    return out.reshape(B, S, D)


# speedup vs baseline: 1.0115x; 1.0115x over previous
import jax
import jax.numpy as jnp
from jax import lax
from jax.experimental import pallas as pl
from jax.experimental.pallas import tpu as pltpu
from jax.experimental.pallas import tpu_sc as plsc

_B, _S, _D = 4, 8192, 1024
_NC, _NS = 2, 16
_NW = _NC * _NS
_RPW = _B * _S // _NW
_WPB = _S // _RPW
_C = 32
_NCHUNK = _RPW // _C
_LANES = 16


def _body(x_hbm, idx_hbm, out_hbm, idx_v, buf0, buf1, buf2,
          gsem0, gsem1, gsem2, ssem0, ssem1, ssem2):
    wid = lax.axis_index("s") * _NC + lax.axis_index("c")
    b = wid // _WPB
    seq_lo = (wid % _WPB) * _RPW
    out_lo = wid * _RPW

    pltpu.sync_copy(idx_hbm.at[pl.ds(seq_lo, _RPW)], idx_v)
    row_off = b * _S
    for j in range(_RPW // _LANES):
        sl = pl.ds(j * _LANES, _LANES)
        idx_v[sl] = idx_v[sl] + row_off

    bufs = (buf0, buf1, buf2)
    gsems = (gsem0, gsem1, gsem2)
    ssems = (ssem0, ssem1, ssem2)

    def _gather_desc(g, par):
        off = pl.multiple_of(g * _C, _C)
        src = x_hbm.at[idx_v.at[pl.ds(off, _C)]]
        return pltpu.make_async_copy(src, bufs[par], gsems[par])

    def _store_desc(g, par):
        dst = out_hbm.at[pl.ds(out_lo + g * _C, _C)]
        return pltpu.make_async_copy(bufs[par], dst, ssems[par])

    _gather_desc(0, 0).start()
    _gather_desc(1, 1).start()
    _gather_desc(2, 2).start()

    for h in range(2):
        _gather_desc(h, h).wait()
        _store_desc(h, h).start()

    @pl.loop(2, _NCHUNK - 3, step=3)
    def _ring(h0):
        for j in range(3):
            h = h0 + j
            par = (2 + j) % 3
            q = (par + 1) % 3
            _store_desc(h - 2, q).wait()
            _gather_desc(h + 1, q).start()
            _gather_desc(h, par).wait()
            _store_desc(h, par).start()

    for h in range(_NCHUNK - 3, _NCHUNK - 1):
        par = h % 3
        q = (par + 1) % 3
        _store_desc(h - 2, q).wait()
        _gather_desc(h + 1, q).start()
        _gather_desc(h, par).wait()
        _store_desc(h, par).start()
    h = _NCHUNK - 1
    _gather_desc(h, h % 3).wait()
    _store_desc(h, h % 3).start()
    for g in range(_NCHUNK - 3, _NCHUNK):
        _store_desc(g, g % 3).wait()


def _build():
    mesh = plsc.VectorSubcoreMesh(
        core_axis_name="c", subcore_axis_name="s",
        num_cores=_NC, num_subcores=_NS)
    return pl.kernel(
        _body,
        out_type=jax.ShapeDtypeStruct((_B * _S, _D), jnp.float32),
        mesh=mesh,
        scratch_types=[
            pltpu.VMEM((_RPW,), jnp.int32),
            pltpu.VMEM((_C, _D), jnp.float32),
            pltpu.VMEM((_C, _D), jnp.float32),
            pltpu.VMEM((_C, _D), jnp.float32),
            pltpu.SemaphoreType.DMA,
            pltpu.SemaphoreType.DMA,
            pltpu.SemaphoreType.DMA,
            pltpu.SemaphoreType.DMA,
            pltpu.SemaphoreType.DMA,
            pltpu.SemaphoreType.DMA,
        ],
    )


def kernel(x, index):
    B, S, D = x.shape
    assert (B, S, D) == (_B, _S, _D)
    out = _build()(x.reshape(B * S, D), index.astype(jnp.int32))
    return out.reshape(B, S, D)


# speedup vs baseline: 1.0183x; 1.0067x over previous
import jax
import jax.numpy as jnp
from jax import lax
from jax.experimental import pallas as pl
from jax.experimental.pallas import tpu as pltpu
from jax.experimental.pallas import tpu_sc as plsc

_B, _S, _D = 4, 8192, 1024
_NC, _NS = 2, 16
_NW = _NC * _NS
_RPW = _B * _S // _NW
_WPB = _S // _RPW
_C = 32
_NCHUNK = _RPW // _C
_LANES = 16


def _body(x_hbm, idx_hbm, out_hbm, idx_v, buf0, buf1,
          gsem0, gsem1, ssem0, ssem1):
    wid = lax.axis_index("s") * _NC + lax.axis_index("c")
    b = wid // _WPB
    seq_lo = (wid % _WPB) * _RPW
    out_lo = wid * _RPW

    pltpu.sync_copy(idx_hbm.at[pl.ds(seq_lo, _RPW)], idx_v)
    row_off = b * _S
    for j in range(_RPW // _LANES):
        sl = pl.ds(j * _LANES, _LANES)
        idx_v[sl] = idx_v[sl] + row_off

    bufs = (buf0, buf1)
    gsems = (gsem0, gsem1)
    ssems = (ssem0, ssem1)

    def _gather_desc(g, par):
        off = pl.multiple_of(g * _C, _C)
        src = x_hbm.at[idx_v.at[pl.ds(off, _C)]]
        return pltpu.make_async_copy(src, bufs[par], gsems[par])

    def _store_desc(g, par):
        dst = out_hbm.at[pl.ds(out_lo + g * _C, _C)]
        return pltpu.make_async_copy(bufs[par], dst, ssems[par])

    _gather_desc(0, 0).start()
    _gather_desc(1, 1).start()

    @pl.loop(0, _NCHUNK - 2, step=2)
    def _chunk_pair(g0):
        for par in range(2):
            g = g0 + par
            _gather_desc(g, par).wait()
            st = _store_desc(g, par)
            st.start()
            st.wait()
            _gather_desc(g + 2, par).start()

    for par in range(2):
        g = _NCHUNK - 2 + par
        _gather_desc(g, par).wait()
        _store_desc(g, par).start()
    for par in range(2):
        _store_desc(_NCHUNK - 2 + par, par).wait()


def _build():
    mesh = plsc.VectorSubcoreMesh(
        core_axis_name="c", subcore_axis_name="s",
        num_cores=_NC, num_subcores=_NS)
    return pl.kernel(
        _body,
        out_type=jax.ShapeDtypeStruct((_B * _S, _D), jnp.float32),
        mesh=mesh,
        scratch_types=[
            pltpu.VMEM((_RPW,), jnp.int32),
            pltpu.VMEM((_C, _D), jnp.float32),
            pltpu.VMEM((_C, _D), jnp.float32),
            pltpu.SemaphoreType.DMA,
            pltpu.SemaphoreType.DMA,
            pltpu.SemaphoreType.DMA,
            pltpu.SemaphoreType.DMA,
        ],
    )


def kernel(x, index):
    B, S, D = x.shape
    assert (B, S, D) == (_B, _S, _D)
    out = _build()(x.reshape(B * S, D), index.astype(jnp.int32))
    return out.reshape(B, S, D)


# speedup vs baseline: 1.0223x; 1.0040x over previous
import jax
import jax.numpy as jnp
from jax import lax
from jax.experimental import pallas as pl
from jax.experimental.pallas import tpu as pltpu
from jax.experimental.pallas import tpu_sc as plsc

_B, _S, _D = 4, 8192, 1024
_NC, _NS = 2, 16
_NW = _NC * _NS
_RPW = _B * _S // _NW
_WPB = _S // _RPW
_C = 32
_NCHUNK = _RPW // _C
_LANES = 16


def _body(x_hbm, idx_hbm, out_hbm, idx_v, buf0, buf1,
          gsem0, gsem1, ssem0, ssem1):
    wid = lax.axis_index("s") * _NC + lax.axis_index("c")
    b = wid // _WPB
    seq_lo = (wid % _WPB) * _RPW
    out_lo = wid * _RPW

    pltpu.sync_copy(idx_hbm.at[pl.ds(seq_lo, _RPW)], idx_v)
    x_batch = x_hbm.at[pl.ds(pl.multiple_of(b * _S, _S), _S)]

    bufs = (buf0, buf1)
    gsems = (gsem0, gsem1)
    ssems = (ssem0, ssem1)

    def _gather_desc(g, par):
        off = pl.multiple_of(g * _C, _C)
        src = x_batch.at[idx_v.at[pl.ds(off, _C)]]
        return pltpu.make_async_copy(src, bufs[par], gsems[par])

    def _store_desc(g, par):
        dst = out_hbm.at[pl.ds(out_lo + g * _C, _C)]
        return pltpu.make_async_copy(bufs[par], dst, ssems[par])

    _gather_desc(0, 0).start()
    _gather_desc(1, 1).start()

    @pl.loop(0, _NCHUNK - 2, step=2)
    def _chunk_pair(g0):
        for par in range(2):
            g = g0 + par
            _gather_desc(g, par).wait()
            st = _store_desc(g, par)
            st.start()
            st.wait()
            _gather_desc(g + 2, par).start()

    for par in range(2):
        g = _NCHUNK - 2 + par
        _gather_desc(g, par).wait()
        _store_desc(g, par).start()
    for par in range(2):
        _store_desc(_NCHUNK - 2 + par, par).wait()


def _build():
    mesh = plsc.VectorSubcoreMesh(
        core_axis_name="c", subcore_axis_name="s",
        num_cores=_NC, num_subcores=_NS)
    return pl.kernel(
        _body,
        out_type=jax.ShapeDtypeStruct((_B * _S, _D), jnp.float32),
        mesh=mesh,
        scratch_types=[
            pltpu.VMEM((_RPW,), jnp.int32),
            pltpu.VMEM((_C, _D), jnp.float32),
            pltpu.VMEM((_C, _D), jnp.float32),
            pltpu.SemaphoreType.DMA,
            pltpu.SemaphoreType.DMA,
            pltpu.SemaphoreType.DMA,
            pltpu.SemaphoreType.DMA,
        ],
    )


def kernel(x, index):
    B, S, D = x.shape
    assert (B, S, D) == (_B, _S, _D)
    out = _build()(x.reshape(B * S, D), index.astype(jnp.int32))
    return out.reshape(B, S, D)
